# Initial kernel scaffold; baseline (speedup 1.0000x reference)
import jax
import jax.numpy as jnp
from jax.experimental import pallas as pl


def kernel(h, r, t, ent_re, ent_im, rel_re, rel_im):
    raise NotImplementedError("write your pallas kernel here")


# speedup vs baseline: 2.1184x; 2.1184x over previous
import functools

import jax
import jax.numpy as jnp
from jax import lax
from jax.experimental import pallas as pl
from jax.experimental.pallas import tpu as pltpu
from jax.experimental.pallas import tpu_sc as plsc

BATCH = 16384
DIM = 128
NC = 2
NS = 16
NW = NC * NS
BPW = BATCH // NW
CH = 64
NCHUNK = BPW // CH
LANES = 16
NSLICE = DIM // LANES


def _complex_score_body(h_hbm, r_hbm, t_hbm, ent_re, ent_im, rel_re, rel_im,
                        out_hbm, idx_h, idx_r, idx_t, hr, hi, tr, ti, rr, ri,
                        out_v, sem):
    wid = lax.axis_index("s") * NC + lax.axis_index("c")
    base = wid * BPW

    pltpu.sync_copy(h_hbm.at[pl.ds(base, BPW)], idx_h)
    pltpu.sync_copy(r_hbm.at[pl.ds(base, BPW)], idx_r)
    pltpu.sync_copy(t_hbm.at[pl.ds(base, BPW)], idx_t)

    lane_iota = lax.iota(jnp.int32, LANES)

    def chunk_body(g, carry):
        sl = pl.ds(g * CH, CH)
        copies = [
            pltpu.async_copy(ent_re.at[idx_h.at[sl]], hr, sem),
            pltpu.async_copy(ent_im.at[idx_h.at[sl]], hi, sem),
            pltpu.async_copy(ent_re.at[idx_t.at[sl]], tr, sem),
            pltpu.async_copy(ent_im.at[idx_t.at[sl]], ti, sem),
            pltpu.async_copy(rel_re.at[idx_r.at[sl]], rr, sem),
            pltpu.async_copy(rel_im.at[idx_r.at[sl]], ri, sem),
        ]
        for c in copies:
            c.wait()

        def group_body(gi, carry2):
            vec = jnp.zeros((LANES,), jnp.float32)
            for j in range(LANES):
                i = gi * LANES + j
                acc = jnp.zeros((LANES,), jnp.float32)
                for s in range(NSLICE):
                    dsl = pl.ds(s * LANES, LANES)
                    a = hr[i, dsl]
                    b = hi[i, dsl]
                    cc = tr[i, dsl]
                    dd = ti[i, dsl]
                    e = rr[i, dsl]
                    f = ri[i, dsl]
                    acc = acc + e * (a * cc + b * dd) + f * (a * dd - b * cc)
                vec = jnp.where(lane_iota == j, jnp.sum(acc), vec)
            out_v[pl.ds(g * CH + gi * LANES, LANES)] = vec
            return carry2

        lax.fori_loop(0, CH // LANES, group_body, 0)
        return carry

    lax.fori_loop(0, NCHUNK, chunk_body, 0)

    pltpu.sync_copy(out_v, out_hbm.at[pl.ds(base, BPW)])


@jax.jit
def _complex_score(h, r, t, ent_re, ent_im, rel_re, rel_im):
    mesh = plsc.VectorSubcoreMesh(core_axis_name="c", subcore_axis_name="s")
    kfn = pl.kernel(
        _complex_score_body,
        out_type=jax.ShapeDtypeStruct((BATCH,), jnp.float32),
        mesh=mesh,
        compiler_params=pltpu.CompilerParams(needs_layout_passes=False),
        scratch_types=[
            pltpu.VMEM((BPW,), jnp.int32),
            pltpu.VMEM((BPW,), jnp.int32),
            pltpu.VMEM((BPW,), jnp.int32),
            pltpu.VMEM((CH, DIM), jnp.float32),
            pltpu.VMEM((CH, DIM), jnp.float32),
            pltpu.VMEM((CH, DIM), jnp.float32),
            pltpu.VMEM((CH, DIM), jnp.float32),
            pltpu.VMEM((CH, DIM), jnp.float32),
            pltpu.VMEM((CH, DIM), jnp.float32),
            pltpu.VMEM((BPW,), jnp.float32),
            pltpu.SemaphoreType.DMA,
        ],
    )
    return kfn(h, r, t, ent_re, ent_im, rel_re, rel_im)


def kernel(h, r, t, ent_re, ent_im, rel_re, rel_im):
    return _complex_score(h.astype(jnp.int32), r.astype(jnp.int32),
                          t.astype(jnp.int32), ent_re, ent_im, rel_re, rel_im)
